# baseline (speedup 1.0000x reference)
import functools

import jax
import jax.numpy as jnp
from jax import lax
from jax.experimental import pallas as pl
from jax.experimental.pallas import tpu as pltpu

_BLK = 8192
_LANE = 128
_SUB = 8
_TILE = _SUB * _LANE


def _zero_argmax_body(x_ref, zero_ref, idx_ref, rmax_ref, ridx_ref, *, nv, v):
    j = pl.program_id(0)

    @pl.when(j == 0)
    def _():
        rmax_ref[...] = jnp.full(rmax_ref.shape, -jnp.inf, rmax_ref.dtype)
        ridx_ref[...] = jnp.zeros(ridx_ref.shape, ridx_ref.dtype)

    zero_ref[...] = jnp.zeros(zero_ref.shape, zero_ref.dtype)

    x = x_ref[...]
    col = lax.broadcasted_iota(jnp.int32, x.shape, 1) + j * x.shape[1]
    x = jnp.where(col < v, x, -jnp.inf)
    m = jnp.max(x, axis=1, keepdims=True)
    lidx = jnp.min(jnp.where(x == m, col, v), axis=1, keepdims=True)
    better = m > rmax_ref[...]
    ridx_ref[...] = jnp.where(better, lidx, ridx_ref[...])
    rmax_ref[...] = jnp.where(better, m, rmax_ref[...])

    @pl.when(j == nv - 1)
    def _():
        idx_ref[...] = ridx_ref[...]


def _scatter_body(tile_ref, cand_ref, zero_ref, out_ref):
    del zero_ref
    i = pl.program_id(0)
    base = tile_ref[i] * _TILE
    sub = lax.broadcasted_iota(jnp.int32, out_ref.shape, 0)
    lane = lax.broadcasted_iota(jnp.int32, out_ref.shape, 1)
    target = base + sub * _LANE + lane
    hit = (target == cand_ref[i, 0]) | (target == cand_ref[i, 1])
    hit = hit | (target == cand_ref[i, 2])
    out_ref[...] = hit.astype(out_ref.dtype)


def kernel(logits):
    b, v = logits.shape
    nv = pl.cdiv(v, _BLK)
    assert v >= _TILE and (b * v) % _TILE == 0

    zeros, idx = pl.pallas_call(
        functools.partial(_zero_argmax_body, nv=nv, v=v),
        grid=(nv,),
        in_specs=[pl.BlockSpec((b, _BLK), lambda i: (0, i))],
        out_specs=[
            pl.BlockSpec((b, _BLK), lambda i: (0, i)),
            pl.BlockSpec((b, 1), lambda i: (0, 0)),
        ],
        out_shape=[
            jax.ShapeDtypeStruct((b, v), logits.dtype),
            jax.ShapeDtypeStruct((b, 1), jnp.int32),
        ],
        scratch_shapes=[
            pltpu.VMEM((b, 1), jnp.float32),
            pltpu.VMEM((b, 1), jnp.int32),
        ],
    )(logits)

    flat = idx[:, 0] + jnp.arange(b, dtype=jnp.int32) * v
    tile = flat // _TILE
    prev_f = jnp.concatenate([flat[:1], flat[:-1]])
    next_f = jnp.concatenate([flat[1:], flat[-1:]])
    cand = jnp.stack([prev_f, flat, next_f], axis=1)

    zeros_flat = zeros.reshape(b * v // _LANE, _LANE)

    grid_spec = pltpu.PrefetchScalarGridSpec(
        num_scalar_prefetch=2,
        grid=(b,),
        in_specs=[
            pl.BlockSpec(memory_space=pl.ANY),
        ],
        out_specs=pl.BlockSpec(
            (_SUB, _LANE), lambda i, tile_pref, cand_pref: (tile_pref[i], 0)
        ),
    )
    out = pl.pallas_call(
        _scatter_body,
        grid_spec=grid_spec,
        out_shape=jax.ShapeDtypeStruct(zeros_flat.shape, logits.dtype),
        input_output_aliases={2: 0},
        compiler_params=pltpu.CompilerParams(
            dimension_semantics=("arbitrary",),
        ),
    )(tile, cand, zeros_flat)
    return out.reshape(b, v)


# speedup vs baseline: 70.1593x; 70.1593x over previous
import functools

import jax
import jax.numpy as jnp
from jax import lax
from jax.experimental import pallas as pl
from jax.experimental.pallas import tpu as pltpu

_BLK = 8192
_LANE = 128
_SUB = 8
_CBLK = 1024


def _zero_argmax_body(x_ref, zero_ref, idx_ref, rmax_ref, ridx_ref, *, nv, v):
    j = pl.program_id(0)

    @pl.when(j == 0)
    def _():
        rmax_ref[...] = jnp.full(rmax_ref.shape, -jnp.inf, rmax_ref.dtype)
        ridx_ref[...] = jnp.zeros(ridx_ref.shape, ridx_ref.dtype)

    zero_ref[...] = jnp.zeros(zero_ref.shape, zero_ref.dtype)

    x = x_ref[...]
    col = lax.broadcasted_iota(jnp.int32, x.shape, 1) + j * x.shape[1]
    x = jnp.where(col < v, x, -jnp.inf)
    m = jnp.max(x, axis=1, keepdims=True)
    lidx = jnp.min(jnp.where(x == m, col, v), axis=1, keepdims=True)
    better = m > rmax_ref[...]
    ridx_ref[...] = jnp.where(better, lidx, ridx_ref[...])
    rmax_ref[...] = jnp.where(better, m, rmax_ref[...])

    @pl.when(j == nv - 1)
    def _():
        idx_ref[...] = ridx_ref[...]


def _scatter_body(ct_ref, idx_ref, zero_ref, out_ref):
    del zero_ref
    i = pl.program_id(0)
    g = i // _SUB
    col = ct_ref[i] * _CBLK + lax.broadcasted_iota(jnp.int32, out_ref.shape, 1)
    sub = lax.broadcasted_iota(jnp.int32, out_ref.shape, 0)
    hit = jnp.zeros(out_ref.shape, jnp.bool_)
    for s in range(_SUB):
        hit = hit | ((sub == s) & (col == idx_ref[g * _SUB + s]))
    out_ref[...] = hit.astype(out_ref.dtype)


def kernel(logits):
    b, v = logits.shape
    nv = pl.cdiv(v, _BLK)
    assert b % _SUB == 0 and v >= _CBLK

    zeros, idx = pl.pallas_call(
        functools.partial(_zero_argmax_body, nv=nv, v=v),
        grid=(nv,),
        in_specs=[pl.BlockSpec((b, _BLK), lambda i: (0, i))],
        out_specs=[
            pl.BlockSpec((b, _BLK), lambda i: (0, i)),
            pl.BlockSpec((b, 1), lambda i: (0, 0)),
        ],
        out_shape=[
            jax.ShapeDtypeStruct((b, v), logits.dtype),
            jax.ShapeDtypeStruct((b, 1), jnp.int32),
        ],
        scratch_shapes=[
            pltpu.VMEM((b, 1), jnp.float32),
            pltpu.VMEM((b, 1), jnp.int32),
        ],
    )(logits)

    col_idx = idx[:, 0]
    col_tile = col_idx // _CBLK

    grid_spec = pltpu.PrefetchScalarGridSpec(
        num_scalar_prefetch=2,
        grid=(b,),
        in_specs=[
            pl.BlockSpec(memory_space=pl.ANY),
        ],
        out_specs=pl.BlockSpec(
            (_SUB, _CBLK), lambda i, ct, ix: (i // _SUB, ct[i])
        ),
    )
    out = pl.pallas_call(
        _scatter_body,
        grid_spec=grid_spec,
        out_shape=jax.ShapeDtypeStruct((b, v), logits.dtype),
        input_output_aliases={2: 0},
        compiler_params=pltpu.CompilerParams(
            dimension_semantics=("arbitrary",),
        ),
    )(col_tile, col_idx, zeros)
    return out


# speedup vs baseline: 81.2368x; 1.1579x over previous
import functools

import jax
import jax.numpy as jnp
from jax import lax
from jax.experimental import pallas as pl
from jax.experimental.pallas import tpu as pltpu

_BLK = 16384
_LANE = 128
_SUB = 8
_CBLK = 1024


def _zero_argmax_body(x_ref, zero_ref, idx_ref, rmax_ref, ridx_ref, *, nv, v):
    j = pl.program_id(0)

    @pl.when(j == 0)
    def _():
        rmax_ref[...] = jnp.full(rmax_ref.shape, -jnp.inf, rmax_ref.dtype)
        ridx_ref[...] = jnp.zeros(ridx_ref.shape, ridx_ref.dtype)

    zero_ref[...] = jnp.zeros(zero_ref.shape, zero_ref.dtype)

    x = x_ref[...]
    col = lax.broadcasted_iota(jnp.int32, x.shape, 1) + j * x.shape[1]
    x = jnp.where(col < v, x, -jnp.inf)
    m = jnp.max(x, axis=1, keepdims=True)
    lidx = jnp.min(jnp.where(x == m, col, v), axis=1, keepdims=True)
    better = m > rmax_ref[...]
    ridx_ref[...] = jnp.where(better, lidx, ridx_ref[...])
    rmax_ref[...] = jnp.where(better, m, rmax_ref[...])

    @pl.when(j == nv - 1)
    def _():
        idx_ref[...] = ridx_ref[...]


def _scatter_body(ct_ref, idx_ref, zero_ref, out_ref):
    del zero_ref
    i = pl.program_id(0)
    g = i // _SUB
    col = ct_ref[i] * _CBLK + lax.broadcasted_iota(jnp.int32, out_ref.shape, 1)
    sub = lax.broadcasted_iota(jnp.int32, out_ref.shape, 0)
    hit = jnp.zeros(out_ref.shape, jnp.bool_)
    for s in range(_SUB):
        hit = hit | ((sub == s) & (col == idx_ref[g * _SUB + s]))
    out_ref[...] = hit.astype(out_ref.dtype)


def kernel(logits):
    b, v = logits.shape
    nv = pl.cdiv(v, _BLK)
    assert b % _SUB == 0 and v >= _CBLK

    zeros, idx = pl.pallas_call(
        functools.partial(_zero_argmax_body, nv=nv, v=v),
        grid=(nv,),
        in_specs=[pl.BlockSpec((b, _BLK), lambda i: (0, i))],
        out_specs=[
            pl.BlockSpec((b, _BLK), lambda i: (0, i)),
            pl.BlockSpec((b, 1), lambda i: (0, 0)),
        ],
        out_shape=[
            jax.ShapeDtypeStruct((b, v), logits.dtype),
            jax.ShapeDtypeStruct((b, 1), jnp.int32),
        ],
        scratch_shapes=[
            pltpu.VMEM((b, 1), jnp.float32),
            pltpu.VMEM((b, 1), jnp.int32),
        ],
    )(logits)

    col_idx = idx[:, 0]
    col_tile = col_idx // _CBLK

    grid_spec = pltpu.PrefetchScalarGridSpec(
        num_scalar_prefetch=2,
        grid=(b,),
        in_specs=[
            pl.BlockSpec(memory_space=pl.ANY),
        ],
        out_specs=pl.BlockSpec(
            (_SUB, _CBLK), lambda i, ct, ix: (i // _SUB, ct[i])
        ),
    )
    out = pl.pallas_call(
        _scatter_body,
        grid_spec=grid_spec,
        out_shape=jax.ShapeDtypeStruct((b, v), logits.dtype),
        input_output_aliases={2: 0},
        compiler_params=pltpu.CompilerParams(
            dimension_semantics=("arbitrary",),
        ),
    )(col_tile, col_idx, zeros)
    return out


# speedup vs baseline: 84.7734x; 1.0435x over previous
import functools

import jax
import jax.numpy as jnp
from jax import lax
from jax.experimental import pallas as pl
from jax.experimental.pallas import tpu as pltpu

_BLK = 32768
_LANE = 128
_SUB = 8
_CBLK = 1024


def _zero_argmax_body(x_ref, zero_ref, idx_ref, rmax_ref, ridx_ref, *, nv, v):
    j = pl.program_id(0)

    @pl.when(j == 0)
    def _():
        rmax_ref[...] = jnp.full(rmax_ref.shape, -jnp.inf, rmax_ref.dtype)
        ridx_ref[...] = jnp.zeros(ridx_ref.shape, ridx_ref.dtype)

    zero_ref[...] = jnp.zeros(zero_ref.shape, zero_ref.dtype)

    x = x_ref[...]
    col = lax.broadcasted_iota(jnp.int32, x.shape, 1) + j * x.shape[1]
    x = jnp.where(col < v, x, -jnp.inf)
    m = jnp.max(x, axis=1, keepdims=True)
    lidx = jnp.min(jnp.where(x == m, col, v), axis=1, keepdims=True)
    better = m > rmax_ref[...]
    ridx_ref[...] = jnp.where(better, lidx, ridx_ref[...])
    rmax_ref[...] = jnp.where(better, m, rmax_ref[...])

    @pl.when(j == nv - 1)
    def _():
        idx_ref[...] = ridx_ref[...]


def _scatter_body(ct_ref, idx_ref, zero_ref, out_ref):
    del zero_ref
    i = pl.program_id(0)
    g = i // _SUB
    col = ct_ref[i] * _CBLK + lax.broadcasted_iota(jnp.int32, out_ref.shape, 1)
    sub = lax.broadcasted_iota(jnp.int32, out_ref.shape, 0)
    hit = jnp.zeros(out_ref.shape, jnp.bool_)
    for s in range(_SUB):
        hit = hit | ((sub == s) & (col == idx_ref[g * _SUB + s]))
    out_ref[...] = hit.astype(out_ref.dtype)


def kernel(logits):
    b, v = logits.shape
    nv = pl.cdiv(v, _BLK)
    assert b % _SUB == 0 and v >= _CBLK

    zeros, idx = pl.pallas_call(
        functools.partial(_zero_argmax_body, nv=nv, v=v),
        grid=(nv,),
        in_specs=[pl.BlockSpec((b, _BLK), lambda i: (0, i))],
        out_specs=[
            pl.BlockSpec((b, _BLK), lambda i: (0, i)),
            pl.BlockSpec((b, 1), lambda i: (0, 0)),
        ],
        out_shape=[
            jax.ShapeDtypeStruct((b, v), logits.dtype),
            jax.ShapeDtypeStruct((b, 1), jnp.int32),
        ],
        scratch_shapes=[
            pltpu.VMEM((b, 1), jnp.float32),
            pltpu.VMEM((b, 1), jnp.int32),
        ],
    )(logits)

    col_idx = idx[:, 0]
    col_tile = col_idx // _CBLK

    grid_spec = pltpu.PrefetchScalarGridSpec(
        num_scalar_prefetch=2,
        grid=(b,),
        in_specs=[
            pl.BlockSpec(memory_space=pl.ANY),
        ],
        out_specs=pl.BlockSpec(
            (_SUB, _CBLK), lambda i, ct, ix: (i // _SUB, ct[i])
        ),
    )
    out = pl.pallas_call(
        _scatter_body,
        grid_spec=grid_spec,
        out_shape=jax.ShapeDtypeStruct((b, v), logits.dtype),
        input_output_aliases={2: 0},
        compiler_params=pltpu.CompilerParams(
            dimension_semantics=("arbitrary",),
        ),
    )(col_tile, col_idx, zeros)
    return out
